# baseline (speedup 1.0000x reference)
import functools

import jax
import jax.numpy as jnp
from jax import lax
from jax.experimental import pallas as pl
from jax.experimental.pallas import tpu as pltpu
from jax.experimental.pallas import tpu_sc as plsc

B = 16384
F = 26
D = 64

NC = 2
NS = 16
NW = NC * NS
BPW = B // NW
CB = 64
NCHUNK = BPW // CB
IDX_PER_CHUNK = CB * F
IDX_ROW = 128
NDMA = IDX_PER_CHUNK // IDX_ROW
IDX_ROWS_PER_W = BPW * F // IDX_ROW


def _pool_body(idx_hbm, table_hbm, out_hbm, idx_v, rows_v, pooled_v, sem):
    wid = lax.axis_index("s") * NC + lax.axis_index("c")
    pltpu.sync_copy(idx_hbm.at[pl.ds(wid * IDX_ROWS_PER_W, IDX_ROWS_PER_W)],
                    idx_v)
    base_row = wid * BPW
    for c in range(NCHUNK):
        cps = []
        for j in range(NDMA):
            cps.append(pltpu.async_copy(
                table_hbm.at[idx_v.at[c * NDMA + j]],
                rows_v.at[pl.ds(j * IDX_ROW, IDX_ROW)],
                sem))
        for cp in cps:
            cp.wait()

        def rbody(r, _):
            base = r * F
            for d in range(D // 16):
                sl = pl.ds(d * 16, 16)
                acc = rows_v[base, sl]
                for f in range(1, F):
                    acc = acc + rows_v[base + f, sl]
                pooled_v[r, sl] = acc
            return _

        lax.fori_loop(0, CB, rbody, None)
        pltpu.sync_copy(pooled_v,
                        out_hbm.at[pl.ds(base_row + c * CB, CB)])


@functools.partial(jax.jit, static_argnames=())
def _sc_gather_pool(idx2d, table):
    mesh = plsc.VectorSubcoreMesh(core_axis_name="c", subcore_axis_name="s")
    return pl.kernel(
        _pool_body,
        out_type=jax.ShapeDtypeStruct((B, D), jnp.float32),
        mesh=mesh,
        scratch_types=[
            pltpu.VMEM((NW * IDX_ROWS_PER_W // NW, IDX_ROW), jnp.int32),
            pltpu.VMEM((IDX_PER_CHUNK, D), jnp.float32),
            pltpu.VMEM((CB, D), jnp.float32),
            pltpu.SemaphoreType.DMA,
        ],
        compiler_params=pltpu.CompilerParams(use_tc_tiling_on_sc=False),
    )(idx2d, table)


BM = 1024


def _mlp_body(x_ref, w1_ref, b1_ref, w2_ref, b2_ref, w3_ref, b3_ref,
              w4_ref, b4_ref, out_ref):
    x = x_ref[...]
    h = jnp.maximum(
        jnp.dot(x, w1_ref[...], preferred_element_type=jnp.float32)
        + b1_ref[...], 0.0)
    h = jnp.maximum(
        jnp.dot(h, w2_ref[...], preferred_element_type=jnp.float32)
        + b2_ref[...], 0.0)
    h = jnp.maximum(
        jnp.dot(h, w3_ref[...], preferred_element_type=jnp.float32)
        + b3_ref[...], 0.0)
    out_ref[...] = jnp.sum(h * w4_ref[...], axis=1) + b4_ref[...]


def _tc_mlp(pooled, W1, b1, W2, b2, W3, b3, W4r, b4):
    full = lambda shape: pl.BlockSpec(shape, lambda i: tuple(0 for _ in shape))
    return pl.pallas_call(
        _mlp_body,
        grid=(B // BM,),
        in_specs=[
            pl.BlockSpec((BM, D), lambda i: (i, 0)),
            full(W1.shape), full(b1.shape),
            full(W2.shape), full(b2.shape),
            full(W3.shape), full(b3.shape),
            full(W4r.shape), full(b4.shape),
        ],
        out_specs=pl.BlockSpec((BM,), lambda i: (i,)),
        out_shape=jax.ShapeDtypeStruct((B,), jnp.float32),
    )(pooled, W1, b1, W2, b2, W3, b3, W4r, b4)


def kernel(feature_ids, table, W1, b1, W2, b2, W3, b3, W4, b4):
    idx2d = feature_ids.reshape(B * F // IDX_ROW, IDX_ROW)
    pooled = _sc_gather_pool(idx2d, table)
    return _tc_mlp(pooled, W1, b1, W2, b2, W3, b3, W4.reshape(1, D), b4)


# speedup vs baseline: 1.0689x; 1.0689x over previous
import functools

import jax
import jax.numpy as jnp
from jax import lax
from jax.experimental import pallas as pl
from jax.experimental.pallas import tpu as pltpu
from jax.experimental.pallas import tpu_sc as plsc

B = 16384
F = 26
D = 64
VOCAB = 1000000

NC = 2
NS = 16
NW = NC * NS
BPW = B // NW
CB = 64
NCHUNK = BPW // CB
IDX_PER_CHUNK = CB * F
IDX_ROW = 128
NDMA = IDX_PER_CHUNK // IDX_ROW
IDX_ROWS_PER_W = BPW * F // IDX_ROW


def _pool_body(idx_hbm, table_hbm, out_hbm, idx_v, rows_v, pooled_v, sem):
    wid = lax.axis_index("s") * NC + lax.axis_index("c")
    pltpu.sync_copy(idx_hbm.at[pl.ds(wid * IDX_ROWS_PER_W, IDX_ROWS_PER_W)],
                    idx_v)
    base_row = wid * BPW
    for c in range(NCHUNK):
        cps = []
        for j in range(NDMA):
            cps.append(pltpu.async_copy(
                table_hbm.at[idx_v.at[c * NDMA + j]],
                rows_v.at[pl.ds(j * IDX_ROW, IDX_ROW)],
                sem))
        for cp in cps:
            cp.wait()

        def rbody(r, _):
            base = r * F
            for d in range(D // 16):
                sl = pl.ds(d * 16, 16)
                acc = rows_v[base, sl]
                for f in range(1, F):
                    acc = acc + rows_v[base + f, sl]
                pooled_v[r, sl] = acc
            return _

        lax.fori_loop(0, CB, rbody, None)
        pltpu.sync_copy(pooled_v,
                        out_hbm.at[pl.ds(base_row + c * CB, CB)])


@functools.partial(jax.jit, static_argnames=())
def _sc_gather_pool(idx2d, table):
    mesh = plsc.VectorSubcoreMesh(core_axis_name="c", subcore_axis_name="s")
    return pl.kernel(
        _pool_body,
        out_type=jax.ShapeDtypeStruct((B, D), jnp.float32),
        mesh=mesh,
        scratch_types=[
            pltpu.VMEM((NW * IDX_ROWS_PER_W // NW, IDX_ROW), jnp.int32),
            pltpu.VMEM((IDX_PER_CHUNK, D), jnp.float32),
            pltpu.VMEM((CB, D), jnp.float32),
            pltpu.SemaphoreType.DMA,
        ],
        compiler_params=pltpu.CompilerParams(use_tc_tiling_on_sc=False),
    )(idx2d, table)


BW = 2048
NBLK = -(-VOCAB // BW)
VH = VOCAB // 2


def _tr_body(x_ref, out_ref):
    xt = x_ref[...].T
    x3 = xt.reshape(BW // 2, 2, D)
    out_ref[...] = jnp.concatenate([x3[:, 0, :], x3[:, 1, :]], axis=1)


def _tc_relayout(tt):
    return pl.pallas_call(
        _tr_body,
        grid=(NBLK,),
        in_specs=[pl.BlockSpec((D, BW), lambda i: (0, i))],
        out_specs=pl.BlockSpec((BW // 2, 2 * D), lambda i: (i, 0)),
        out_shape=jax.ShapeDtypeStruct((VH, 2 * D), jnp.float32),
    )(tt)


BM = 1024


def _mlp_body(x_ref, w1_ref, b1_ref, w2_ref, b2_ref, w3_ref, b3_ref,
              w4_ref, b4_ref, out_ref):
    x = x_ref[...]
    h = jnp.maximum(
        jnp.dot(x, w1_ref[...], preferred_element_type=jnp.float32)
        + b1_ref[...], 0.0)
    h = jnp.maximum(
        jnp.dot(h, w2_ref[...], preferred_element_type=jnp.float32)
        + b2_ref[...], 0.0)
    h = jnp.maximum(
        jnp.dot(h, w3_ref[...], preferred_element_type=jnp.float32)
        + b3_ref[...], 0.0)
    out_ref[...] = jnp.sum(h * w4_ref[...], axis=1) + b4_ref[...]


def _tc_mlp(pooled, W1, b1, W2, b2, W3, b3, W4r, b4):
    full = lambda shape: pl.BlockSpec(shape, lambda i: tuple(0 for _ in shape))
    return pl.pallas_call(
        _mlp_body,
        grid=(B // BM,),
        in_specs=[
            pl.BlockSpec((BM, D), lambda i: (i, 0)),
            full(W1.shape), full(b1.shape),
            full(W2.shape), full(b2.shape),
            full(W3.shape), full(b3.shape),
            full(W4r.shape), full(b4.shape),
        ],
        out_specs=pl.BlockSpec((BM,), lambda i: (i,)),
        out_shape=jax.ShapeDtypeStruct((B,), jnp.float32),
    )(pooled, W1, b1, W2, b2, W3, b3, W4r, b4)


def kernel(feature_ids, table, W1, b1, W2, b2, W3, b3, W4, b4):
    t_lin = _tc_relayout(table.T).reshape(VOCAB, D)
    idx2d = feature_ids.reshape(B * F // IDX_ROW, IDX_ROW)
    pooled = _sc_gather_pool(idx2d, t_lin)
    return _tc_mlp(pooled, W1, b1, W2, b2, W3, b3, W4.reshape(1, D), b4)
